# Initial kernel scaffold; baseline (speedup 1.0000x reference)
import jax
import jax.numpy as jnp
from jax.experimental import pallas as pl


def kernel(xyz):
    raise NotImplementedError("write your pallas kernel here")


# speedup vs baseline: 3.5700x; 3.5700x over previous
import jax
import jax.numpy as jnp
from jax import lax
from jax.experimental import pallas as pl
from jax.experimental.pallas import tpu as pltpu

_G = 128
_K = 32


def _tc_body(x_ref, c_ref, idx_ref, d_ref):
    n = x_ref.shape[2]
    x0 = x_ref[0, 0:1, :]
    x1 = x_ref[0, 1:2, :]
    x2 = x_ref[0, 2:3, :]
    iota_l = lax.broadcasted_iota(jnp.int32, (1, n), 1)
    iota_g = lax.broadcasted_iota(jnp.int32, (_G, 1), 0)

    def fps_step(s, carry):
        distv, far, c0a, c1a, c2a = carry
        oh = iota_l == far
        c0 = jnp.sum(jnp.where(oh, x0, 0.0))
        c1 = jnp.sum(jnp.where(oh, x1, 0.0))
        c2 = jnp.sum(jnp.where(oh, x2, 0.0))
        d0 = x0 - c0
        d1 = x1 - c1
        d2 = x2 - c2
        d = d0 * d0 + d1 * d1 + d2 * d2
        distv = jnp.minimum(distv, d)
        m = jnp.max(distv)
        far_new = jnp.min(jnp.where(distv == m, iota_l, n))
        ohg = iota_g == s
        c0a = jnp.where(ohg, c0, c0a)
        c1a = jnp.where(ohg, c1, c1a)
        c2a = jnp.where(ohg, c2, c2a)
        return distv, far_new, c0a, c1a, c2a

    zg = jnp.zeros((_G, 1), jnp.float32)
    distv0 = jnp.full((1, n), 1e10, jnp.float32)
    _, _, c0a, c1a, c2a = lax.fori_loop(
        0, _G, fps_step, (distv0, jnp.array(0, jnp.int32), zg, zg, zg))

    c_ref[0, :, 0:1] = c0a
    c_ref[0, :, 1:2] = c1a
    c_ref[0, :, 2:3] = c2a

    e0 = c0a - x0
    e1 = c1a - x1
    e2 = c2a - x2
    d_ref[:] = jnp.sqrt(e0 * e0 + e1 * e1 + e2 * e2)

    iota_k = lax.broadcasted_iota(jnp.int32, (_G, _K), 1)

    def topk_step(j, idxacc):
        dm = d_ref[:]
        m = jnp.min(dm, axis=1, keepdims=True)
        sel = jnp.min(jnp.where(dm == m, iota_l, n), axis=1, keepdims=True)
        d_ref[:] = jnp.where(iota_l == sel, jnp.inf, dm)
        return jnp.where(iota_k == j, sel, idxacc)

    idx_ref[0] = lax.fori_loop(
        0, _K, topk_step, jnp.zeros((_G, _K), jnp.int32))


def _run_tc(x_pad, interpret=False):
    b, _, n = x_pad.shape
    return pl.pallas_call(
        _tc_body,
        grid=(b,),
        in_specs=[pl.BlockSpec((1, 8, n), lambda i: (i, 0, 0))],
        out_specs=[
            pl.BlockSpec((1, _G, 8), lambda i: (i, 0, 0)),
            pl.BlockSpec((1, _G, _K), lambda i: (i, 0, 0)),
        ],
        out_shape=[
            jax.ShapeDtypeStruct((b, _G, 8), jnp.float32),
            jax.ShapeDtypeStruct((b, _G, _K), jnp.int32),
        ],
        scratch_shapes=[pltpu.VMEM((_G, n), jnp.float32)],
        interpret=interpret,
    )(x_pad)


def kernel(xyz):
    b, n, c = xyz.shape
    x_t = jnp.transpose(xyz, (0, 2, 1))
    x_pad = jnp.concatenate(
        [x_t, jnp.zeros((b, 8 - c, n), xyz.dtype)], axis=1)
    c_pad, idx = _run_tc(x_pad)
    center = c_pad[:, :, :3]
    flat = idx.reshape(b, _G * _K)
    patch = jnp.take_along_axis(xyz, flat[:, :, None], axis=1)
    patch = patch.reshape(b, _G, _K, c) - center[:, :, None, :]
    return (patch, center)
